# baseline (speedup 1.0000x reference)
import functools

import jax
import jax.numpy as jnp
from jax import lax
from jax.experimental import pallas as pl
from jax.experimental.pallas import tpu as pltpu
from jax.experimental.pallas import tpu_sc as plsc

N_NODES_K = 10000
N_EDGES_K = 320000
D_FEAT_K = 128
SCALE = 1.0 / (32.0 ** 0.5)

NUM_CORES = 2
NUM_SUBCORES = 16
NUM_TILES = NUM_CORES * NUM_SUBCORES
WINDOW = 128
N_WINDOWS = N_EDGES_K // WINDOW
MAX_W_PER_TILE = -(-N_WINDOWS // NUM_TILES)
NBUF = 2

ROWS_PER_TILE = 624
ROWS_TAIL = N_NODES_K - NUM_SUBCORES * ROWS_PER_TILE
TAIL_ROW0 = NUM_SUBCORES * ROWS_PER_TILE


def _sc_scatter_partials(x, idx2d):
    mesh = plsc.VectorSubcoreMesh(core_axis_name="c", subcore_axis_name="s")

    @functools.partial(
        pl.kernel,
        out_type=[jax.ShapeDtypeStruct((N_NODES_K, D_FEAT_K), jnp.float32),
                  jax.ShapeDtypeStruct((N_NODES_K, D_FEAT_K), jnp.float32)],
        mesh=mesh,
        scratch_types=[
            pltpu.VMEM_SHARED((N_NODES_K, D_FEAT_K), jnp.float32),
            pltpu.VMEM((NBUF, WINDOW), jnp.int32),
            pltpu.VMEM((WINDOW, D_FEAT_K), jnp.float32),
            pltpu.VMEM((WINDOW, D_FEAT_K), jnp.float32),
            pltpu.SemaphoreType.DMA((NBUF,)),
            pltpu.SemaphoreType.DMA,
        ],
    )
    def k(x_hbm, idx_hbm, out0_hbm, out1_hbm, acc_sp, idx_v, x_v0, x_v1,
          ld_sem, z_sem):
        c = lax.axis_index("c")
        s = lax.axis_index("s")
        wid = s * NUM_CORES + c
        xs = [x_v0, x_v1]

        @pl.loop(0, WINDOW)
        def _(i):
            for j in range(0, D_FEAT_K, 16):
                x_v1.at[i, pl.ds(j, 16)][...] = jnp.zeros((16,), jnp.float32)

        row0 = s * ROWS_PER_TILE
        n_full = ROWS_PER_TILE // WINDOW
        for i in range(n_full):
            pltpu.async_copy(x_v1, acc_sp.at[pl.ds(row0 + i * WINDOW, WINDOW)],
                             z_sem)
        rem = ROWS_PER_TILE - n_full * WINDOW
        if rem:
            pltpu.async_copy(x_v1.at[pl.ds(0, rem)],
                             acc_sp.at[pl.ds(row0 + ROWS_PER_TILE - rem, rem)],
                             z_sem)

        @pl.when(s == 0)
        def _():
            pltpu.async_copy(x_v1.at[pl.ds(0, ROWS_TAIL)],
                             acc_sp.at[pl.ds(TAIL_ROW0, ROWS_TAIL)], z_sem)

        def g_of(w):
            return w * NUM_TILES + wid

        def start_load_u(w, b):
            g = g_of(w)
            pltpu.async_copy(idx_hbm.at[g], idx_v.at[b], ld_sem.at[b])
            pltpu.async_copy(x_hbm.at[pl.ds(g * WINDOW, WINDOW)],
                             xs[b], ld_sem.at[b])

        def wait_load_u(w, b):
            pltpu.make_async_copy(idx_hbm.at[0], idx_v.at[b],
                                  ld_sem.at[b]).wait()
            pltpu.make_async_copy(x_hbm.at[pl.ds(0, WINDOW)], xs[b],
                                  ld_sem.at[b]).wait()

        def sync_scatter_u(w, b):
            pltpu.sync_copy(xs[b], acc_sp.at[idx_v.at[b]], add=True)

        def start_load(w, b):
            @pl.when(g_of(w) < N_WINDOWS)
            def _():
                start_load_u(w, b)

        def wait_load(w, b):
            @pl.when(g_of(w) < N_WINDOWS)
            def _():
                wait_load_u(w, b)

        def sync_scatter(w, b):
            @pl.when(g_of(w) < N_WINDOWS)
            def _():
                sync_scatter_u(w, b)

        start_load(0, 0)

        for i in range(n_full):
            pltpu.make_async_copy(x_v1, acc_sp.at[pl.ds(0, WINDOW)],
                                  z_sem).wait()
        if rem:
            pltpu.make_async_copy(x_v1.at[pl.ds(0, rem)],
                                  acc_sp.at[pl.ds(0, rem)], z_sem).wait()

        @pl.when(s == 0)
        def _():
            pltpu.make_async_copy(x_v1.at[pl.ds(0, ROWS_TAIL)],
                                  acc_sp.at[pl.ds(0, ROWS_TAIL)], z_sem).wait()

        plsc.subcore_barrier()

        @pl.loop(0, 76, step=NBUF)
        def _(k0):
            for b in range(NBUF):
                w = k0 + b
                nb = (b + 1) % NBUF
                start_load_u(w + 1, nb)
                wait_load_u(w, b)
                sync_scatter_u(w, b)

        start_load_u(77, 1)
        wait_load_u(76, 0)
        sync_scatter_u(76, 0)
        start_load(78, 0)
        wait_load_u(77, 1)
        sync_scatter_u(77, 1)
        wait_load(78, 0)
        sync_scatter(78, 0)

        plsc.subcore_barrier()
        for cc, out_hbm in ((0, out0_hbm), (1, out1_hbm)):
            @pl.when(c == cc)
            def _(out_hbm=out_hbm):
                pltpu.sync_copy(acc_sp.at[pl.ds(row0, ROWS_PER_TILE)],
                                out_hbm.at[pl.ds(row0, ROWS_PER_TILE)])

                @pl.when(s == 0)
                def _():
                    pltpu.sync_copy(acc_sp.at[pl.ds(TAIL_ROW0, ROWS_TAIL)],
                                    out_hbm.at[pl.ds(TAIL_ROW0, ROWS_TAIL)])

    return k(x, idx2d)


def _tc_combine_body(p0_ref, p1_ref, o_ref):
    o_ref[...] = (p0_ref[...] + p1_ref[...]) * SCALE


def _tc_combine(p0, p1):
    blk = 400
    spec = pl.BlockSpec((blk, D_FEAT_K), lambda i: (i, 0))
    return pl.pallas_call(
        _tc_combine_body,
        grid=(N_NODES_K // blk,),
        in_specs=[spec, spec],
        out_specs=spec,
        out_shape=jax.ShapeDtypeStruct((N_NODES_K, D_FEAT_K), jnp.float32),
    )(p0, p1)


@jax.jit
def kernel(x, index):
    idx2d = index.astype(jnp.int32).reshape(N_WINDOWS, WINDOW)
    p0, p1 = _sc_scatter_partials(x, idx2d)
    return _tc_combine(p0, p1)


# speedup vs baseline: 1.0687x; 1.0687x over previous
import functools

import jax
import jax.numpy as jnp
from jax import lax
from jax.experimental import pallas as pl
from jax.experimental.pallas import tpu as pltpu
from jax.experimental.pallas import tpu_sc as plsc

N_NODES_K = 10000
N_EDGES_K = 320000
D_FEAT_K = 128
SCALE = 1.0 / (32.0 ** 0.5)

NUM_CORES = 2
NUM_SUBCORES = 16
NUM_TILES = NUM_CORES * NUM_SUBCORES
WINDOW = 128
N_WINDOWS = N_EDGES_K // WINDOW
MAX_W_PER_TILE = -(-N_WINDOWS // NUM_TILES)
NBUF = 2

ROWS_PER_TILE = 624
ROWS_TAIL = N_NODES_K - NUM_SUBCORES * ROWS_PER_TILE
TAIL_ROW0 = NUM_SUBCORES * ROWS_PER_TILE


def _sc_scatter_partials(x, idx2d):
    mesh = plsc.VectorSubcoreMesh(core_axis_name="c", subcore_axis_name="s")

    @functools.partial(
        pl.kernel,
        out_type=[jax.ShapeDtypeStruct((N_NODES_K, D_FEAT_K), jnp.float32),
                  jax.ShapeDtypeStruct((N_NODES_K, D_FEAT_K), jnp.float32)],
        mesh=mesh,
        scratch_types=[
            pltpu.VMEM_SHARED((N_NODES_K, D_FEAT_K), jnp.float32),
            pltpu.VMEM((NBUF, WINDOW), jnp.int32),
            pltpu.VMEM((WINDOW, D_FEAT_K), jnp.float32),
            pltpu.VMEM((WINDOW, D_FEAT_K), jnp.float32),
            pltpu.SemaphoreType.DMA((NBUF,)),
            pltpu.SemaphoreType.DMA,
        ],
    )
    def k(x_hbm, idx_hbm, out0_hbm, out1_hbm, acc_sp, idx_v, x_v0, x_v1,
          ld_sem, z_sem):
        c = lax.axis_index("c")
        s = lax.axis_index("s")
        wid = s * NUM_CORES + c
        xs = [x_v0, x_v1]

        @pl.loop(0, WINDOW)
        def _(i):
            for j in range(0, D_FEAT_K, 16):
                x_v1.at[i, pl.ds(j, 16)][...] = jnp.zeros((16,), jnp.float32)

        row0 = s * ROWS_PER_TILE
        n_full = ROWS_PER_TILE // WINDOW
        for i in range(n_full):
            pltpu.async_copy(x_v1, acc_sp.at[pl.ds(row0 + i * WINDOW, WINDOW)],
                             z_sem)
        rem = ROWS_PER_TILE - n_full * WINDOW
        if rem:
            pltpu.async_copy(x_v1.at[pl.ds(0, rem)],
                             acc_sp.at[pl.ds(row0 + ROWS_PER_TILE - rem, rem)],
                             z_sem)

        @pl.when(s == 0)
        def _():
            pltpu.async_copy(x_v1.at[pl.ds(0, ROWS_TAIL)],
                             acc_sp.at[pl.ds(TAIL_ROW0, ROWS_TAIL)], z_sem)

        def g_of(w):
            return w * NUM_TILES + wid

        def start_load_u(w, b):
            g = g_of(w)
            pltpu.async_copy(idx_hbm.at[g], idx_v.at[b], ld_sem.at[b])
            pltpu.async_copy(x_hbm.at[pl.ds(g * WINDOW, WINDOW)],
                             xs[b], ld_sem.at[b])

        def wait_load_u(w, b):
            pltpu.make_async_copy(idx_hbm.at[0], idx_v.at[b],
                                  ld_sem.at[b]).wait()
            pltpu.make_async_copy(x_hbm.at[pl.ds(0, WINDOW)], xs[b],
                                  ld_sem.at[b]).wait()

        def sync_scatter_u(w, b):
            pltpu.sync_copy(xs[b], acc_sp.at[idx_v.at[b]], add=True)

        def start_load(w, b):
            @pl.when(g_of(w) < N_WINDOWS)
            def _():
                start_load_u(w, b)

        def wait_load(w, b):
            @pl.when(g_of(w) < N_WINDOWS)
            def _():
                wait_load_u(w, b)

        def sync_scatter(w, b):
            @pl.when(g_of(w) < N_WINDOWS)
            def _():
                sync_scatter_u(w, b)

        start_load(0, 0)

        for i in range(n_full):
            pltpu.make_async_copy(x_v1, acc_sp.at[pl.ds(0, WINDOW)],
                                  z_sem).wait()
        if rem:
            pltpu.make_async_copy(x_v1.at[pl.ds(0, rem)],
                                  acc_sp.at[pl.ds(0, rem)], z_sem).wait()

        @pl.when(s == 0)
        def _():
            pltpu.make_async_copy(x_v1.at[pl.ds(0, ROWS_TAIL)],
                                  acc_sp.at[pl.ds(0, ROWS_TAIL)], z_sem).wait()

        plsc.subcore_barrier()

        @pl.loop(0, 76, step=NBUF)
        def _(k0):
            for b in range(NBUF):
                w = k0 + b
                nb = (b + 1) % NBUF
                start_load_u(w + 1, nb)
                wait_load_u(w, b)
                sync_scatter_u(w, b)

        start_load_u(77, 1)
        wait_load_u(76, 0)
        sync_scatter_u(76, 0)
        start_load(78, 0)
        wait_load_u(77, 1)
        sync_scatter_u(77, 1)
        wait_load(78, 0)
        sync_scatter(78, 0)

        plsc.subcore_barrier()
        for cc, out_hbm in ((0, out0_hbm), (1, out1_hbm)):
            @pl.when(c == cc)
            def _(out_hbm=out_hbm):
                pltpu.sync_copy(acc_sp.at[pl.ds(row0, ROWS_PER_TILE)],
                                out_hbm.at[pl.ds(row0, ROWS_PER_TILE)])

                @pl.when(s == 0)
                def _():
                    pltpu.sync_copy(acc_sp.at[pl.ds(TAIL_ROW0, ROWS_TAIL)],
                                    out_hbm.at[pl.ds(TAIL_ROW0, ROWS_TAIL)])

    return k(x, idx2d)


def _tc_combine_body(p0_ref, p1_ref, o_ref):
    o_ref[...] = (p0_ref[...] + p1_ref[...]) * SCALE


def _tc_combine(p0, p1):
    blk = 2000
    spec = pl.BlockSpec((blk, D_FEAT_K), lambda i: (i, 0))
    return pl.pallas_call(
        _tc_combine_body,
        grid=(N_NODES_K // blk,),
        in_specs=[spec, spec],
        out_specs=spec,
        out_shape=jax.ShapeDtypeStruct((N_NODES_K, D_FEAT_K), jnp.float32),
    )(p0, p1)


@jax.jit
def kernel(x, index):
    idx2d = index.astype(jnp.int32).reshape(N_WINDOWS, WINDOW)
    p0, p1 = _sc_scatter_partials(x, idx2d)
    return _tc_combine(p0, p1)


# speedup vs baseline: 1.0745x; 1.0055x over previous
import functools

import jax
import jax.numpy as jnp
from jax import lax
from jax.experimental import pallas as pl
from jax.experimental.pallas import tpu as pltpu
from jax.experimental.pallas import tpu_sc as plsc

N_NODES_K = 10000
N_EDGES_K = 320000
D_FEAT_K = 128
SCALE = 1.0 / (32.0 ** 0.5)

NUM_CORES = 2
NUM_SUBCORES = 16
NUM_TILES = NUM_CORES * NUM_SUBCORES
WINDOW = 128
N_WINDOWS = N_EDGES_K // WINDOW
MAX_W_PER_TILE = -(-N_WINDOWS // NUM_TILES)
NBUF = 2

ROWS_PER_TILE = 624
ROWS_TAIL = N_NODES_K - NUM_SUBCORES * ROWS_PER_TILE
TAIL_ROW0 = NUM_SUBCORES * ROWS_PER_TILE


def _sc_scatter_partials(x, idx2d):
    mesh = plsc.VectorSubcoreMesh(core_axis_name="c", subcore_axis_name="s")

    @functools.partial(
        pl.kernel,
        out_type=[jax.ShapeDtypeStruct((N_NODES_K, D_FEAT_K), jnp.float32),
                  jax.ShapeDtypeStruct((N_NODES_K, D_FEAT_K), jnp.float32)],
        mesh=mesh,
        scratch_types=[
            pltpu.VMEM_SHARED((N_NODES_K, D_FEAT_K), jnp.float32),
            pltpu.VMEM((NBUF, WINDOW), jnp.int32),
            pltpu.VMEM((WINDOW, D_FEAT_K), jnp.float32),
            pltpu.VMEM((WINDOW, D_FEAT_K), jnp.float32),
            pltpu.SemaphoreType.DMA((NBUF,)),
            pltpu.SemaphoreType.DMA,
        ],
    )
    def k(x_hbm, idx_hbm, out0_hbm, out1_hbm, acc_sp, idx_v, x_v0, x_v1,
          ld_sem, z_sem):
        c = lax.axis_index("c")
        s = lax.axis_index("s")
        wid = s * NUM_CORES + c
        xs = [x_v0, x_v1]

        @pl.loop(0, WINDOW)
        def _(i):
            for j in range(0, D_FEAT_K, 16):
                x_v1.at[i, pl.ds(j, 16)][...] = jnp.zeros((16,), jnp.float32)

        row0 = s * ROWS_PER_TILE
        n_full = ROWS_PER_TILE // WINDOW
        for i in range(n_full):
            pltpu.async_copy(x_v1, acc_sp.at[pl.ds(row0 + i * WINDOW, WINDOW)],
                             z_sem)
        rem = ROWS_PER_TILE - n_full * WINDOW
        if rem:
            pltpu.async_copy(x_v1.at[pl.ds(0, rem)],
                             acc_sp.at[pl.ds(row0 + ROWS_PER_TILE - rem, rem)],
                             z_sem)

        @pl.when(s == 0)
        def _():
            pltpu.async_copy(x_v1.at[pl.ds(0, ROWS_TAIL)],
                             acc_sp.at[pl.ds(TAIL_ROW0, ROWS_TAIL)], z_sem)

        def g_of(w):
            return w * NUM_TILES + wid

        def start_load_u(w, b):
            g = g_of(w)
            pltpu.async_copy(idx_hbm.at[g], idx_v.at[b], ld_sem.at[b])
            pltpu.async_copy(x_hbm.at[pl.ds(g * WINDOW, WINDOW)],
                             xs[b], ld_sem.at[b])

        def wait_load_u(w, b):
            pltpu.make_async_copy(idx_hbm.at[0], idx_v.at[b],
                                  ld_sem.at[b]).wait()
            pltpu.make_async_copy(x_hbm.at[pl.ds(0, WINDOW)], xs[b],
                                  ld_sem.at[b]).wait()

        def sync_scatter_u(w, b):
            pltpu.sync_copy(xs[b], acc_sp.at[idx_v.at[b]], add=True)

        def start_load(w, b):
            @pl.when(g_of(w) < N_WINDOWS)
            def _():
                start_load_u(w, b)

        def wait_load(w, b):
            @pl.when(g_of(w) < N_WINDOWS)
            def _():
                wait_load_u(w, b)

        def sync_scatter(w, b):
            @pl.when(g_of(w) < N_WINDOWS)
            def _():
                sync_scatter_u(w, b)

        start_load(0, 0)

        for i in range(n_full):
            pltpu.make_async_copy(x_v1, acc_sp.at[pl.ds(0, WINDOW)],
                                  z_sem).wait()
        if rem:
            pltpu.make_async_copy(x_v1.at[pl.ds(0, rem)],
                                  acc_sp.at[pl.ds(0, rem)], z_sem).wait()

        @pl.when(s == 0)
        def _():
            pltpu.make_async_copy(x_v1.at[pl.ds(0, ROWS_TAIL)],
                                  acc_sp.at[pl.ds(0, ROWS_TAIL)], z_sem).wait()

        plsc.subcore_barrier()

        @pl.loop(0, 76, step=NBUF)
        def _(k0):
            for b in range(NBUF):
                w = k0 + b
                nb = (b + 1) % NBUF
                start_load_u(w + 1, nb)
                wait_load_u(w, b)
                sync_scatter_u(w, b)

        start_load_u(77, 1)
        wait_load_u(76, 0)
        sync_scatter_u(76, 0)
        start_load(78, 0)
        wait_load_u(77, 1)
        sync_scatter_u(77, 1)
        wait_load(78, 0)
        sync_scatter(78, 0)

        plsc.subcore_barrier()
        for cc, out_hbm in ((0, out0_hbm), (1, out1_hbm)):
            @pl.when(c == cc)
            def _(out_hbm=out_hbm):
                pltpu.sync_copy(acc_sp.at[pl.ds(row0, ROWS_PER_TILE)],
                                out_hbm.at[pl.ds(row0, ROWS_PER_TILE)])

                @pl.when(s == 0)
                def _():
                    pltpu.sync_copy(acc_sp.at[pl.ds(TAIL_ROW0, ROWS_TAIL)],
                                    out_hbm.at[pl.ds(TAIL_ROW0, ROWS_TAIL)])

    return k(x, idx2d)


def _tc_combine_body(p0_ref, p1_ref, o_ref):
    o_ref[...] = (p0_ref[...] + p1_ref[...]) * SCALE


def _tc_combine(p0, p1):
    blk = 10000
    spec = pl.BlockSpec((blk, D_FEAT_K), lambda i: (i, 0))
    return pl.pallas_call(
        _tc_combine_body,
        grid=(N_NODES_K // blk,),
        in_specs=[spec, spec],
        out_specs=spec,
        out_shape=jax.ShapeDtypeStruct((N_NODES_K, D_FEAT_K), jnp.float32),
    )(p0, p1)


@jax.jit
def kernel(x, index):
    idx2d = index.astype(jnp.int32).reshape(N_WINDOWS, WINDOW)
    p0, p1 = _sc_scatter_partials(x, idx2d)
    return _tc_combine(p0, p1)
